# baseline (speedup 1.0000x reference)
import jax
import jax.numpy as jnp
from jax.experimental import pallas as pl
import jax.experimental.pallas.tpu as pltpu

_B = 8
_CIN = 1792
_CO = 112
_S = 56
_HW = _S * _S
_KB = 8
_KC = _CIN // _KB
_JB = 8
_RB = _HW // _JB


def _cprep_body(c_ref, o_ref):
    cw = c_ref[...]
    cn = jnp.sum(cw * cw, axis=0, keepdims=True)
    o_ref[0:_CO, :] = -2.0 * cw
    o_ref[_CO:_CO + 8, :] = jnp.concatenate(
        [cn, jnp.zeros((7, _HW), jnp.float32)], axis=0)
    o_ref[_CO + 8:, :] = jnp.zeros((8, _HW), jnp.float32)


def _fused_body(x_ref, w_ref, b_ref, ca_ref, phi_ref, sc_ref, acc_ref):
    k = pl.program_id(1)
    x2 = x_ref[0].reshape(_KC, _HW)
    r = jax.lax.dot_general(w_ref[0], x2, (((1,), (0,)), ((), ())),
                            preferred_element_type=jnp.float32)

    @pl.when(k == 0)
    def _():
        acc_ref[...] = r

    @pl.when(k > 0)
    def _():
        acc_ref[...] += r

    @pl.when(k == _KB - 1)
    def _():
        _tail(b_ref, ca_ref, phi_ref, sc_ref, acc_ref)


def _tail(b_ref, ca_ref, phi_ref, sc_ref, acc_ref):
    acc = acc_ref[...]
    iota = jax.lax.broadcasted_iota(jnp.int32, (1, _HW), 1)
    wpos = iota % _S
    zero = jnp.float32(0.0)
    lt = pltpu.roll(acc, 1, 1)
    rt = pltpu.roll(acc, _HW - 1, 1)
    rw = (acc + jnp.where(wpos == 0, zero, lt)
          + jnp.where(wpos == _S - 1, zero, rt))
    u2 = pltpu.roll(rw, _S, 1)
    d2 = pltpu.roll(rw, _HW - _S, 1)
    rh = (rw + jnp.where(iota < _S, zero, u2)
          + jnp.where(iota >= _HW - _S, zero, d2))
    phit = rh * jnp.float32(1.0 / 9.0) + b_ref[...]

    phit_pad = jnp.concatenate(
        [phit, jnp.ones((1, _HW), jnp.float32),
         jnp.zeros((15, _HW), jnp.float32)], axis=0)
    phi128 = phit_pad.T
    phi_ref[0] = phi128[:, :_CO]

    ca = ca_ref[...]
    big = jnp.float32(1e30)
    for j in range(_JB):
        ph = phi128[j * _RB:(j + 1) * _RB, :]
        rn = jnp.sum(ph[:, :_CO] * ph[:, :_CO], axis=1, keepdims=True)
        e = jax.lax.dot_general(ph, ca, (((1,), (0,)), ((), ())),
                                preferred_element_type=jnp.float32)
        m1 = jnp.min(e, axis=1, keepdims=True)
        e2 = jnp.where(e > m1, e, big)
        m2 = jnp.min(e2, axis=1, keepdims=True)
        e3 = jnp.where(e2 > m2, e2, big)
        m3 = jnp.min(e3, axis=1, keepdims=True)
        d1 = jnp.sqrt(jnp.maximum(m1 + rn, 0.0))
        d2_ = jnp.sqrt(jnp.maximum(m2 + rn, 0.0))
        d3 = jnp.sqrt(jnp.maximum(m3 + rn, 0.0))
        sc_ref[0, pl.ds(j * _RB, _RB), :] = d1 / (
            1.0 + jnp.exp(d1 - d2_) + jnp.exp(d1 - d3))


def kernel(p, W, bias, C):
    x = p.reshape(_B, _CIN, _S, _S)
    w4 = W.reshape(_CO, _KB, _KC).transpose(1, 0, 2)
    brow = bias[:, None]

    caug = pl.pallas_call(
        _cprep_body,
        grid=(1,),
        in_specs=[pl.BlockSpec((_CO, _HW), lambda i: (0, 0))],
        out_specs=pl.BlockSpec((128, _HW), lambda i: (0, 0)),
        out_shape=jax.ShapeDtypeStruct((128, _HW), jnp.float32),
    )(C)

    phi_p, score = pl.pallas_call(
        _fused_body,
        grid=(_B, _KB),
        in_specs=[
            pl.BlockSpec((1, _KC, _S, _S), lambda b, k: (b, k, 0, 0)),
            pl.BlockSpec((1, _CO, _KC), lambda b, k: (k, 0, 0)),
            pl.BlockSpec((_CO, 1), lambda b, k: (0, 0)),
            pl.BlockSpec((128, _HW), lambda b, k: (0, 0)),
        ],
        out_specs=[
            pl.BlockSpec((1, _HW, _CO), lambda b, k: (b, 0, 0)),
            pl.BlockSpec((1, _HW, 1), lambda b, k: (b, 0, 0)),
        ],
        out_shape=[
            jax.ShapeDtypeStruct((_B, _HW, _CO), jnp.float32),
            jax.ShapeDtypeStruct((_B, _HW, 1), jnp.float32),
        ],
        scratch_shapes=[pltpu.VMEM((_CO, _HW), jnp.float32)],
        compiler_params=pltpu.CompilerParams(
            vmem_limit_bytes=100 * 1024 * 1024),
    )(x, w4, brow, caug)

    return (score.reshape(_B, 1, _S, _S), phi_p)
